# baseline (speedup 1.0000x reference)
import functools

import jax
import jax.numpy as jnp
from jax import lax
from jax.experimental import pallas as pl
from jax.experimental.pallas import tpu as pltpu
from jax.experimental.pallas import tpu_sc as plsc

HID = 2048
NEXP = 8
TOPK = 2
T = 8192
TM = 128
MTILES = T * TOPK // TM + NEXP
CAP = MTILES * TM

NC, NS, L = 2, 16, 16
NW = NC * NS
TPW = T // NW
CHUNK = 16
NCHUNK = TPW // CHUNK


_RM = 512


def _router_body(x_ref, gw_ref, gb_ref, logits_ref, sel_ref, w_ref, cnt_ref):
    i = pl.program_id(0)
    x = x_ref[...]
    logits = jnp.dot(x, gw_ref[...], preferred_element_type=jnp.float32)
    logits = logits + gb_ref[...]
    logits_ref[...] = logits

    iota = lax.broadcasted_iota(jnp.int32, (_RM, NEXP), 1)
    a1 = jnp.argmax(logits, axis=1).astype(jnp.int32)
    oh1 = (iota == a1[:, None])
    masked = jnp.where(oh1, jnp.full_like(logits, -jnp.inf), logits)
    a2 = jnp.argmax(masked, axis=1).astype(jnp.int32)
    oh2 = (iota == a2[:, None])

    m = jnp.max(logits, axis=1, keepdims=True)
    p = jnp.exp(logits - m)
    p = p / jnp.sum(p, axis=1, keepdims=True)
    p1 = jnp.sum(jnp.where(oh1, p, 0.0), axis=1)
    p2 = jnp.sum(jnp.where(oh2, p, 0.0), axis=1)
    s = p1 + p2
    sel_ref[...] = jnp.stack([a1, a2], axis=1)
    w_ref[...] = jnp.stack([p1 / s, p2 / s], axis=1)

    cnt = jnp.sum((oh1 | oh2).astype(jnp.int32), axis=0, keepdims=True)

    @pl.when(i == 0)
    def _():
        cnt_ref[...] = cnt

    @pl.when(i > 0)
    def _():
        cnt_ref[...] = cnt_ref[...] + cnt


def _router(x2d, gate_W, gate_b):
    return pl.pallas_call(
        _router_body,
        grid=(T // _RM,),
        in_specs=[
            pl.BlockSpec((_RM, HID), lambda i: (i, 0)),
            pl.BlockSpec((HID, NEXP), lambda i: (0, 0)),
            pl.BlockSpec((1, NEXP), lambda i: (0, 0)),
        ],
        out_specs=[
            pl.BlockSpec((_RM, NEXP), lambda i: (i, 0)),
            pl.BlockSpec((_RM, TOPK), lambda i: (i, 0)),
            pl.BlockSpec((_RM, TOPK), lambda i: (i, 0)),
            pl.BlockSpec((1, NEXP), lambda i: (0, 0)),
        ],
        out_shape=[
            jax.ShapeDtypeStruct((T, NEXP), jnp.float32),
            jax.ShapeDtypeStruct((T, TOPK), jnp.int32),
            jax.ShapeDtypeStruct((T, TOPK), jnp.float32),
            jax.ShapeDtypeStruct((1, NEXP), jnp.int32),
        ],
    )(x2d, gate_W, gate_b.reshape(1, NEXP))


_PM = 512


def _pos_body(sel_ref, cnt_ref, p0_ref, p1_ref, te_ref, carry_ref):
    i = pl.program_id(0)

    @pl.when(i == 0)
    def _():
        carry_ref[...] = jnp.zeros((1, NEXP), jnp.int32)

    cnt = cnt_ref[...]
    padded = ((cnt + TM - 1) // TM) * TM
    r8 = lax.broadcasted_iota(jnp.int32, (NEXP, NEXP), 0)
    c8 = lax.broadcasted_iota(jnp.int32, (NEXP, NEXP), 1)
    tri8 = (r8 < c8).astype(jnp.float32)
    off = jnp.dot(padded.astype(jnp.float32), tri8,
                  preferred_element_type=jnp.float32)

    @pl.when(i == 0)
    def _():
        csum = off + padded.astype(jnp.float32)
        starts = (lax.broadcasted_iota(jnp.int32, (MTILES, NEXP), 0)
                  * TM).astype(jnp.float32)
        ends = jnp.broadcast_to(csum, (MTILES, NEXP))
        te = jnp.sum((ends <= starts).astype(jnp.int32), axis=1)
        te_ref[...] = jnp.minimum(te, NEXP - 1).reshape(1, MTILES)

    sel = sel_ref[...]
    ioE = lax.broadcasted_iota(jnp.int32, (_PM, NEXP), 1)
    oh0 = (sel[:, 0:1] == ioE).astype(jnp.float32)
    oh1 = (sel[:, 1:2] == ioE).astype(jnp.float32)
    ohsum = oh0 + oh1
    rT = lax.broadcasted_iota(jnp.int32, (_PM, _PM), 0)
    cT = lax.broadcasted_iota(jnp.int32, (_PM, _PM), 1)
    triT = (rT > cT).astype(jnp.float32)
    prior = jnp.dot(triT, ohsum, preferred_element_type=jnp.float32)
    base = off + carry_ref[...].astype(jnp.float32)
    pos0 = jnp.sum((base + prior) * oh0, axis=1)
    pos1 = jnp.sum((base + prior) * oh1, axis=1)
    p0_ref[...] = pos0.astype(jnp.int32).reshape(1, 1, _PM)
    p1_ref[...] = pos1.astype(jnp.int32).reshape(1, 1, _PM)
    carry_ref[...] = carry_ref[...] + jnp.sum(
        ohsum, axis=0, keepdims=True).astype(jnp.int32)


def _positions(sel, counts):
    nblk = T // _PM
    return pl.pallas_call(
        _pos_body,
        grid=(nblk,),
        in_specs=[
            pl.BlockSpec((_PM, TOPK), lambda i: (i, 0)),
            pl.BlockSpec((1, NEXP), lambda i: (0, 0)),
        ],
        out_specs=[
            pl.BlockSpec((1, 1, _PM), lambda i: (i, 0, 0)),
            pl.BlockSpec((1, 1, _PM), lambda i: (i, 0, 0)),
            pl.BlockSpec((1, MTILES), lambda i: (0, 0)),
        ],
        out_shape=[
            jax.ShapeDtypeStruct((nblk, 1, _PM), jnp.int32),
            jax.ShapeDtypeStruct((nblk, 1, _PM), jnp.int32),
            jax.ShapeDtypeStruct((1, MTILES), jnp.int32),
        ],
        scratch_shapes=[pltpu.VMEM((1, NEXP), jnp.int32)],
    )(sel, counts)


@functools.lru_cache(maxsize=1)
def _sc_mesh():
    return plsc.VectorSubcoreMesh(core_axis_name="c", subcore_axis_name="s",
                                  num_cores=NC, num_subcores=NS)


def _scatter_body(x_hbm, i0_hbm, i1_hbm, w0_hbm, w1_hbm, xs_hbm, ws_hbm,
                  i0_v, i1_v, w0_v, w1_v, rows_a, rows_b, sem_a, sem_b):
    wid = lax.axis_index("s") * NC + lax.axis_index("c")
    base_tok = wid * TPW
    pltpu.sync_copy(i0_hbm.at[wid], i0_v)
    pltpu.sync_copy(i1_hbm.at[wid], i1_v)
    pltpu.sync_copy(w0_hbm.at[wid], w0_v)
    pltpu.sync_copy(w1_hbm.at[wid], w1_v)
    bufs = (rows_a, rows_b)
    sems = (sem_a, sem_b)
    pend = [None, None]
    pltpu.sync_copy(x_hbm.at[pl.ds(base_tok, CHUNK)], rows_a)
    for c in range(NCHUNK):
        b = c % 2
        nb = (c + 1) % 2
        pend[b] = [
            pltpu.async_copy(bufs[b], xs_hbm.at[i0_v.at[c]], sems[b]),
            pltpu.async_copy(bufs[b], xs_hbm.at[i1_v.at[c]], sems[b]),
            pltpu.async_copy(w0_v.at[c], ws_hbm.at[i0_v.at[c]], sems[b]),
            pltpu.async_copy(w1_v.at[c], ws_hbm.at[i1_v.at[c]], sems[b]),
        ]
        if c + 1 < NCHUNK:
            if pend[nb] is not None:
                for d in pend[nb]:
                    d.wait()
                pend[nb] = None
            pltpu.sync_copy(
                x_hbm.at[pl.ds(base_tok + (c + 1) * CHUNK, CHUNK)], bufs[nb])
    for p in pend:
        if p is not None:
            for d in p:
                d.wait()


def _scatter_sc(x2d, i0, i1, w0, w1):
    return pl.kernel(
        _scatter_body,
        out_type=[
            jax.ShapeDtypeStruct((CAP, HID), jnp.float32),
            jax.ShapeDtypeStruct((CAP,), jnp.float32),
        ],
        mesh=_sc_mesh(),
        scratch_types=[
            pltpu.VMEM((NCHUNK, CHUNK), jnp.int32),
            pltpu.VMEM((NCHUNK, CHUNK), jnp.int32),
            pltpu.VMEM((NCHUNK, CHUNK), jnp.float32),
            pltpu.VMEM((NCHUNK, CHUNK), jnp.float32),
            pltpu.VMEM((CHUNK, HID), jnp.float32),
            pltpu.VMEM((CHUNK, HID), jnp.float32),
            pltpu.SemaphoreType.DMA,
            pltpu.SemaphoreType.DMA,
        ],
    )(x2d, i0, i1, w0, w1)


_GN = 1024


def _gmm_body(te_ref, xs_ref, w_ref, b_ref, ws_ref, y_ref):
    xb = xs_ref[...].astype(jnp.bfloat16)
    wb = w_ref[0].astype(jnp.bfloat16)
    y = jnp.dot(xb, wb, preferred_element_type=jnp.float32)
    y_ref[...] = (y + b_ref[0]) * ws_ref[...]


def _grouped_matmul(te, xs, expert_Wb, expert_b, wsorted):
    return pl.pallas_call(
        _gmm_body,
        grid_spec=pltpu.PrefetchScalarGridSpec(
            num_scalar_prefetch=1,
            grid=(MTILES,),
            in_specs=[
                pl.BlockSpec((TM, HID), lambda m, te_r: (m, 0)),
                pl.BlockSpec((1, HID, HID), lambda m, te_r: (te_r[m], 0, 0)),
                pl.BlockSpec((1, 1, HID), lambda m, te_r: (te_r[m], 0, 0)),
                pl.BlockSpec((TM, 1), lambda m, te_r: (m, 0)),
            ],
            out_specs=pl.BlockSpec((TM, HID), lambda m, te_r: (m, 0)),
        ),
        out_shape=jax.ShapeDtypeStruct((CAP, HID), jnp.float32),
        compiler_params=pltpu.CompilerParams(
            dimension_semantics=("arbitrary",),
        ),
    )(te, xs, expert_Wb, expert_b.reshape(NEXP, 1, HID),
      wsorted.reshape(CAP, 1))


CCH = 8
NCC = TPW // CCH


def _combine_body(y_hbm, i0_hbm, i1_hbm, out_hbm,
                  i0_v, i1_v, r0a, r1a, r0b, r1b, sem_a, sem_b):
    wid = lax.axis_index("s") * NC + lax.axis_index("c")
    base_tok = wid * TPW
    pltpu.sync_copy(i0_hbm.at[wid], i0_v)
    pltpu.sync_copy(i1_hbm.at[wid], i1_v)
    bufs = ((r0a, r1a), (r0b, r1b))
    sems = (sem_a, sem_b)

    def issue(c, b):
        return [
            pltpu.async_copy(y_hbm.at[i0_v.at[c]], bufs[b][0], sems[b]),
            pltpu.async_copy(y_hbm.at[i1_v.at[c]], bufs[b][1], sems[b]),
        ]

    pend = [None, None]
    pend[0] = issue(0, 0)
    for c in range(NCC):
        b = c % 2
        nb = (c + 1) % 2
        if c + 1 < NCC:
            pend[nb] = issue(c + 1, nb)
        for d in pend[b]:
            d.wait()
        pend[b] = None
        r0, r1 = bufs[b]

        def row_body(r, _):
            def vec_body(j, _):
                v = r1[r, pl.ds(j * L, L)]
                plsc.addupdate(r0.at[r, pl.ds(j * L, L)], v)
                return 0

            return lax.fori_loop(0, HID // L, vec_body, 0, unroll=8)

        lax.fori_loop(0, CCH, row_body, 0)
        pltpu.sync_copy(r0, out_hbm.at[pl.ds(base_tok + c * CCH, CCH)])


def _combine_sc(y, i0, i1):
    return pl.kernel(
        _combine_body,
        out_type=jax.ShapeDtypeStruct((T, HID), jnp.float32),
        mesh=_sc_mesh(),
        scratch_types=[
            pltpu.VMEM((NCC, CCH), jnp.int32),
            pltpu.VMEM((NCC, CCH), jnp.int32),
            pltpu.VMEM((CCH, HID), jnp.float32),
            pltpu.VMEM((CCH, HID), jnp.float32),
            pltpu.VMEM((CCH, HID), jnp.float32),
            pltpu.VMEM((CCH, HID), jnp.float32),
            pltpu.SemaphoreType.DMA,
            pltpu.SemaphoreType.DMA,
        ],
    )(y, i0, i1)


def kernel(x, gate_W, gate_b, expert_W, expert_b):
    b, s, d = x.shape
    x2d = x.reshape(-1, d)
    logits, sel, w, counts = _router(x2d, gate_W, gate_b)
    p0, p1, te = _positions(sel, counts)
    i0 = p0.reshape(NW, NCHUNK, CHUNK)
    i1 = p1.reshape(NW, NCHUNK, CHUNK)
    w0 = w[:, 0].reshape(NW, NCHUNK, CHUNK)
    w1 = w[:, 1].reshape(NW, NCHUNK, CHUNK)
    xs, wsorted = _scatter_sc(x2d, i0, i1, w0, w1)
    y = _grouped_matmul(te.reshape(MTILES), xs, expert_W, expert_b, wsorted)
    out = _combine_sc(y, p0.reshape(NW, NCC, CCH), p1.reshape(NW, NCC, CCH))
    return out.reshape(b, s, d), logits, sel


# speedup vs baseline: 1.0620x; 1.0620x over previous
import functools

import jax
import jax.numpy as jnp
from jax import lax
from jax.experimental import pallas as pl
from jax.experimental.pallas import tpu as pltpu
from jax.experimental.pallas import tpu_sc as plsc

HID = 2048
NEXP = 8
TOPK = 2
T = 8192
TM = 256
MTILES = T * TOPK // TM + NEXP
CAP = MTILES * TM

NC, NS, L = 2, 16, 16
NW = NC * NS
TPW = T // NW
CHUNK = 16
NCHUNK = TPW // CHUNK


_RM = 512


def _router_body(x_ref, gw_ref, gb_ref, logits_ref, sel_ref, w_ref, cnt_ref):
    i = pl.program_id(0)
    x = x_ref[...]
    logits = jnp.dot(x, gw_ref[...], preferred_element_type=jnp.float32)
    logits = logits + gb_ref[...]
    logits_ref[...] = logits

    iota = lax.broadcasted_iota(jnp.int32, (_RM, NEXP), 1)
    a1 = jnp.argmax(logits, axis=1).astype(jnp.int32)
    oh1 = (iota == a1[:, None])
    masked = jnp.where(oh1, jnp.full_like(logits, -jnp.inf), logits)
    a2 = jnp.argmax(masked, axis=1).astype(jnp.int32)
    oh2 = (iota == a2[:, None])

    m = jnp.max(logits, axis=1, keepdims=True)
    p = jnp.exp(logits - m)
    p = p / jnp.sum(p, axis=1, keepdims=True)
    p1 = jnp.sum(jnp.where(oh1, p, 0.0), axis=1)
    p2 = jnp.sum(jnp.where(oh2, p, 0.0), axis=1)
    s = p1 + p2
    sel_ref[...] = jnp.stack([a1, a2], axis=1)
    w_ref[...] = jnp.stack([p1 / s, p2 / s], axis=1)

    cnt = jnp.sum((oh1 | oh2).astype(jnp.int32), axis=0, keepdims=True)

    @pl.when(i == 0)
    def _():
        cnt_ref[...] = cnt

    @pl.when(i > 0)
    def _():
        cnt_ref[...] = cnt_ref[...] + cnt


def _router(x2d, gate_W, gate_b):
    return pl.pallas_call(
        _router_body,
        grid=(T // _RM,),
        in_specs=[
            pl.BlockSpec((_RM, HID), lambda i: (i, 0)),
            pl.BlockSpec((HID, NEXP), lambda i: (0, 0)),
            pl.BlockSpec((1, NEXP), lambda i: (0, 0)),
        ],
        out_specs=[
            pl.BlockSpec((_RM, NEXP), lambda i: (i, 0)),
            pl.BlockSpec((_RM, TOPK), lambda i: (i, 0)),
            pl.BlockSpec((_RM, TOPK), lambda i: (i, 0)),
            pl.BlockSpec((1, NEXP), lambda i: (0, 0)),
        ],
        out_shape=[
            jax.ShapeDtypeStruct((T, NEXP), jnp.float32),
            jax.ShapeDtypeStruct((T, TOPK), jnp.int32),
            jax.ShapeDtypeStruct((T, TOPK), jnp.float32),
            jax.ShapeDtypeStruct((1, NEXP), jnp.int32),
        ],
    )(x2d, gate_W, gate_b.reshape(1, NEXP))


_PM = 512


def _pos_body(sel_ref, cnt_ref, p0_ref, p1_ref, te_ref, carry_ref):
    i = pl.program_id(0)

    @pl.when(i == 0)
    def _():
        carry_ref[...] = jnp.zeros((1, NEXP), jnp.int32)

    cnt = cnt_ref[...]
    padded = ((cnt + TM - 1) // TM) * TM
    r8 = lax.broadcasted_iota(jnp.int32, (NEXP, NEXP), 0)
    c8 = lax.broadcasted_iota(jnp.int32, (NEXP, NEXP), 1)
    tri8 = (r8 < c8).astype(jnp.float32)
    off = jnp.dot(padded.astype(jnp.float32), tri8,
                  preferred_element_type=jnp.float32)

    @pl.when(i == 0)
    def _():
        csum = off + padded.astype(jnp.float32)
        starts = (lax.broadcasted_iota(jnp.int32, (MTILES, NEXP), 0)
                  * TM).astype(jnp.float32)
        ends = jnp.broadcast_to(csum, (MTILES, NEXP))
        te = jnp.sum((ends <= starts).astype(jnp.int32), axis=1)
        te_ref[...] = jnp.minimum(te, NEXP - 1).reshape(1, MTILES)

    sel = sel_ref[...]
    ioE = lax.broadcasted_iota(jnp.int32, (_PM, NEXP), 1)
    oh0 = (sel[:, 0:1] == ioE).astype(jnp.float32)
    oh1 = (sel[:, 1:2] == ioE).astype(jnp.float32)
    ohsum = oh0 + oh1
    rT = lax.broadcasted_iota(jnp.int32, (_PM, _PM), 0)
    cT = lax.broadcasted_iota(jnp.int32, (_PM, _PM), 1)
    triT = (rT > cT).astype(jnp.float32)
    prior = jnp.dot(triT, ohsum, preferred_element_type=jnp.float32)
    base = off + carry_ref[...].astype(jnp.float32)
    pos0 = jnp.sum((base + prior) * oh0, axis=1)
    pos1 = jnp.sum((base + prior) * oh1, axis=1)
    p0_ref[...] = pos0.astype(jnp.int32).reshape(1, 1, _PM)
    p1_ref[...] = pos1.astype(jnp.int32).reshape(1, 1, _PM)
    carry_ref[...] = carry_ref[...] + jnp.sum(
        ohsum, axis=0, keepdims=True).astype(jnp.int32)


def _positions(sel, counts):
    nblk = T // _PM
    return pl.pallas_call(
        _pos_body,
        grid=(nblk,),
        in_specs=[
            pl.BlockSpec((_PM, TOPK), lambda i: (i, 0)),
            pl.BlockSpec((1, NEXP), lambda i: (0, 0)),
        ],
        out_specs=[
            pl.BlockSpec((1, 1, _PM), lambda i: (i, 0, 0)),
            pl.BlockSpec((1, 1, _PM), lambda i: (i, 0, 0)),
            pl.BlockSpec((1, MTILES), lambda i: (0, 0)),
        ],
        out_shape=[
            jax.ShapeDtypeStruct((nblk, 1, _PM), jnp.int32),
            jax.ShapeDtypeStruct((nblk, 1, _PM), jnp.int32),
            jax.ShapeDtypeStruct((1, MTILES), jnp.int32),
        ],
        scratch_shapes=[pltpu.VMEM((1, NEXP), jnp.int32)],
    )(sel, counts)


@functools.lru_cache(maxsize=1)
def _sc_mesh():
    return plsc.VectorSubcoreMesh(core_axis_name="c", subcore_axis_name="s",
                                  num_cores=NC, num_subcores=NS)


def _scatter_body(x_hbm, i0_hbm, i1_hbm, w0_hbm, w1_hbm, xs_hbm, ws_hbm,
                  i0_v, i1_v, w0_v, w1_v, rows_a, rows_b, sem_a, sem_b):
    wid = lax.axis_index("s") * NC + lax.axis_index("c")
    base_tok = wid * TPW
    pltpu.sync_copy(i0_hbm.at[wid], i0_v)
    pltpu.sync_copy(i1_hbm.at[wid], i1_v)
    pltpu.sync_copy(w0_hbm.at[wid], w0_v)
    pltpu.sync_copy(w1_hbm.at[wid], w1_v)
    bufs = (rows_a, rows_b)
    sems = (sem_a, sem_b)
    pend = [None, None]
    pltpu.sync_copy(x_hbm.at[pl.ds(base_tok, CHUNK)], rows_a)
    for c in range(NCHUNK):
        b = c % 2
        nb = (c + 1) % 2
        pend[b] = [
            pltpu.async_copy(bufs[b], xs_hbm.at[i0_v.at[c]], sems[b]),
            pltpu.async_copy(bufs[b], xs_hbm.at[i1_v.at[c]], sems[b]),
            pltpu.async_copy(w0_v.at[c], ws_hbm.at[i0_v.at[c]], sems[b]),
            pltpu.async_copy(w1_v.at[c], ws_hbm.at[i1_v.at[c]], sems[b]),
        ]
        if c + 1 < NCHUNK:
            if pend[nb] is not None:
                for d in pend[nb]:
                    d.wait()
                pend[nb] = None
            pltpu.sync_copy(
                x_hbm.at[pl.ds(base_tok + (c + 1) * CHUNK, CHUNK)], bufs[nb])
    for p in pend:
        if p is not None:
            for d in p:
                d.wait()


def _scatter_sc(x2d, i0, i1, w0, w1):
    return pl.kernel(
        _scatter_body,
        out_type=[
            jax.ShapeDtypeStruct((CAP, HID), jnp.float32),
            jax.ShapeDtypeStruct((CAP,), jnp.float32),
        ],
        mesh=_sc_mesh(),
        scratch_types=[
            pltpu.VMEM((NCHUNK, CHUNK), jnp.int32),
            pltpu.VMEM((NCHUNK, CHUNK), jnp.int32),
            pltpu.VMEM((NCHUNK, CHUNK), jnp.float32),
            pltpu.VMEM((NCHUNK, CHUNK), jnp.float32),
            pltpu.VMEM((CHUNK, HID), jnp.float32),
            pltpu.VMEM((CHUNK, HID), jnp.float32),
            pltpu.SemaphoreType.DMA,
            pltpu.SemaphoreType.DMA,
        ],
    )(x2d, i0, i1, w0, w1)


_GN = 1024


def _gmm_body(te_ref, xs_ref, w_ref, b_ref, ws_ref, y_ref):
    xb = xs_ref[...].astype(jnp.bfloat16)
    wb = w_ref[0].astype(jnp.bfloat16)
    y = jnp.dot(xb, wb, preferred_element_type=jnp.float32)
    y_ref[...] = (y + b_ref[0]) * ws_ref[...]


def _grouped_matmul(te, xs, expert_Wb, expert_b, wsorted):
    return pl.pallas_call(
        _gmm_body,
        grid_spec=pltpu.PrefetchScalarGridSpec(
            num_scalar_prefetch=1,
            grid=(MTILES,),
            in_specs=[
                pl.BlockSpec((TM, HID), lambda m, te_r: (m, 0)),
                pl.BlockSpec((1, HID, HID), lambda m, te_r: (te_r[m], 0, 0)),
                pl.BlockSpec((1, 1, HID), lambda m, te_r: (te_r[m], 0, 0)),
                pl.BlockSpec((TM, 1), lambda m, te_r: (m, 0)),
            ],
            out_specs=pl.BlockSpec((TM, HID), lambda m, te_r: (m, 0)),
        ),
        out_shape=jax.ShapeDtypeStruct((CAP, HID), jnp.float32),
        compiler_params=pltpu.CompilerParams(
            dimension_semantics=("arbitrary",),
        ),
    )(te, xs, expert_Wb, expert_b.reshape(NEXP, 1, HID),
      wsorted.reshape(CAP, 1))


CCH = 8
NCC = TPW // CCH


def _combine_body(y_hbm, i0_hbm, i1_hbm, out_hbm,
                  i0_v, i1_v, r0a, r1a, r0b, r1b, sem_a, sem_b):
    wid = lax.axis_index("s") * NC + lax.axis_index("c")
    base_tok = wid * TPW
    pltpu.sync_copy(i0_hbm.at[wid], i0_v)
    pltpu.sync_copy(i1_hbm.at[wid], i1_v)
    bufs = ((r0a, r1a), (r0b, r1b))
    sems = (sem_a, sem_b)

    def issue(c, b):
        return [
            pltpu.async_copy(y_hbm.at[i0_v.at[c]], bufs[b][0], sems[b]),
            pltpu.async_copy(y_hbm.at[i1_v.at[c]], bufs[b][1], sems[b]),
        ]

    pend = [None, None]
    pend[0] = issue(0, 0)
    for c in range(NCC):
        b = c % 2
        nb = (c + 1) % 2
        if c + 1 < NCC:
            pend[nb] = issue(c + 1, nb)
        for d in pend[b]:
            d.wait()
        pend[b] = None
        r0, r1 = bufs[b]

        def row_body(r, _):
            def vec_body(j, _):
                v = r1[r, pl.ds(j * L, L)]
                plsc.addupdate(r0.at[r, pl.ds(j * L, L)], v)
                return 0

            return lax.fori_loop(0, HID // L, vec_body, 0, unroll=8)

        lax.fori_loop(0, CCH, row_body, 0)
        pltpu.sync_copy(r0, out_hbm.at[pl.ds(base_tok + c * CCH, CCH)])


def _combine_sc(y, i0, i1):
    return pl.kernel(
        _combine_body,
        out_type=jax.ShapeDtypeStruct((T, HID), jnp.float32),
        mesh=_sc_mesh(),
        scratch_types=[
            pltpu.VMEM((NCC, CCH), jnp.int32),
            pltpu.VMEM((NCC, CCH), jnp.int32),
            pltpu.VMEM((CCH, HID), jnp.float32),
            pltpu.VMEM((CCH, HID), jnp.float32),
            pltpu.VMEM((CCH, HID), jnp.float32),
            pltpu.VMEM((CCH, HID), jnp.float32),
            pltpu.SemaphoreType.DMA,
            pltpu.SemaphoreType.DMA,
        ],
    )(y, i0, i1)


def kernel(x, gate_W, gate_b, expert_W, expert_b):
    b, s, d = x.shape
    x2d = x.reshape(-1, d)
    logits, sel, w, counts = _router(x2d, gate_W, gate_b)
    p0, p1, te = _positions(sel, counts)
    i0 = p0.reshape(NW, NCHUNK, CHUNK)
    i1 = p1.reshape(NW, NCHUNK, CHUNK)
    w0 = w[:, 0].reshape(NW, NCHUNK, CHUNK)
    w1 = w[:, 1].reshape(NW, NCHUNK, CHUNK)
    xs, wsorted = _scatter_sc(x2d, i0, i1, w0, w1)
    y = _grouped_matmul(te.reshape(MTILES), xs, expert_W, expert_b, wsorted)
    out = _combine_sc(y, p0.reshape(NW, NCC, CCH), p1.reshape(NW, NCC, CCH))
    return out.reshape(b, s, d), logits, sel
